# Initial kernel scaffold; baseline (speedup 1.0000x reference)
import jax
import jax.numpy as jnp
from jax.experimental import pallas as pl


def kernel(feats, emb, gamma1, beta1, W1, b1c, We, be, gamma2, beta2, W2, b2c, batch_idx, nbrs):
    raise NotImplementedError("write your pallas kernel here")


# speedup vs baseline: 13.3526x; 13.3526x over previous
import functools

import jax
import jax.numpy as jnp
from jax import lax
from jax.experimental import pallas as pl
from jax.experimental.pallas import tpu as pltpu
from jax.experimental.pallas import tpu_sc as plsc

N = 200000
C = 64
B = 4
NPB = 50000
G = 32
CG = C // G
K = 27
KP = 28
CK = C * K
TILE = 1000
TPB = NPB // TILE
NT = N // TILE + 1
NP_ROWS = NT * TILE
ZIDX = N * K

STILE = 2000
SPB = NPB // STILE

NC = 2
NS = 16
NW = NC * NS
CH = 32
VPG = 4
IPG = VPG * KP
RG = CH // VPG
PER_W = 6272


def _stats_body(x_ref, s_ref, q_ref):
    t = pl.program_id(1)
    x = x_ref[...]
    s = jnp.sum(x, axis=0)[None, None, :]
    q = jnp.sum(x * x, axis=0)[None, None, :]

    @pl.when(t == 0)
    def _init():
        s_ref[...] = s
        q_ref[...] = q

    @pl.when(t != 0)
    def _acc():
        s_ref[...] += s
        q_ref[...] += q


_stats_call = pl.pallas_call(
    _stats_body,
    grid=(B, SPB),
    in_specs=[pl.BlockSpec((STILE, C), lambda b, t: (b * SPB + t, 0))],
    out_specs=[
        pl.BlockSpec((1, 1, C), lambda b, t: (b, 0, 0)),
        pl.BlockSpec((1, 1, C), lambda b, t: (b, 0, 0)),
    ],
    out_shape=[
        jax.ShapeDtypeStruct((B, 1, C), jnp.float32),
        jax.ShapeDtypeStruct((B, 1, C), jnp.float32),
    ],
)


def _matnorm_body(x_ref, a_ref, c_ref, w_ref, y_ref):
    t = pl.program_id(0)
    x = x_ref[...]
    a = a_ref[0]
    cc = c_ref[0]
    h = x * a + cc
    h = h * jax.nn.sigmoid(h)
    row = t * TILE + lax.broadcasted_iota(jnp.int32, (TILE, 1), 0)
    h = jnp.where(row < N, h, 0.0)
    y_ref[...] = jnp.dot(h, w_ref[...], preferred_element_type=jnp.float32)


_matnorm_call = pl.pallas_call(
    _matnorm_body,
    grid=(NT,),
    in_specs=[
        pl.BlockSpec((TILE, C), lambda t: (jnp.minimum(t, N // TILE - 1), 0)),
        pl.BlockSpec((1, 1, C), lambda t: (jnp.minimum(t // TPB, B - 1), 0, 0)),
        pl.BlockSpec((1, 1, C), lambda t: (jnp.minimum(t // TPB, B - 1), 0, 0)),
        pl.BlockSpec((C, CK), lambda t: (0, 0)),
    ],
    out_specs=pl.BlockSpec((TILE, CK), lambda t: (t, 0)),
    out_shape=jax.ShapeDtypeStruct((NP_ROWS, CK), jnp.float32),
)


def _emb_body(e_ref, w_ref, o_ref):
    e = e_ref[...]
    h = e * jax.nn.sigmoid(e)
    o_ref[...] = jnp.dot(h, w_ref[...], preferred_element_type=jnp.float32)


def _emb_call(emb, We):
    return pl.pallas_call(
        _emb_body,
        out_shape=jax.ShapeDtypeStruct((B, C), jnp.float32),
    )(emb, We)


def _make_gather(residual: bool):
    mesh = plsc.VectorSubcoreMesh(
        core_axis_name="c", subcore_axis_name="s", num_cores=NC,
        num_subcores=NS)

    scratch = [
        pltpu.VMEM((RG, IPG), jnp.int32),
        pltpu.VMEM((RG, IPG, C), jnp.float32),
        pltpu.VMEM((CH, C), jnp.float32),
        pltpu.VMEM((C,), jnp.float32),
        pltpu.VMEM((CH, C), jnp.float32),
        pltpu.SemaphoreType.DMA,
    ]

    def body(tab_ref, idx_ref, bias_ref, *rest):
        if residual:
            res_ref, out_ref, idx_v, rows_v, out_v, bias_v, res_v, sem = rest
        else:
            out_ref, idx_v, rows_v, out_v, bias_v, res_v, sem = rest
        cid = lax.axis_index("c")
        sid = lax.axis_index("s")
        wid = sid * NC + cid
        v0 = wid * PER_W
        rounds = jnp.minimum(PER_W, N - v0) // CH
        pltpu.sync_copy(bias_ref, bias_v)

        @pl.loop(0, rounds)
        def _round(r):
            vbase = pl.multiple_of(v0 + r * CH, CH)
            rbase = pl.multiple_of(vbase // VPG, RG)
            pltpu.sync_copy(idx_ref.at[pl.ds(rbase, RG)], idx_v)
            if residual:
                pltpu.sync_copy(res_ref.at[pl.ds(vbase, CH)], res_v)
            copies = [
                pltpu.async_copy(tab_ref.at[idx_v.at[j]], rows_v.at[j], sem)
                for j in range(RG)
            ]
            for cp in copies:
                cp.wait()
            for j in range(RG):
                for q in range(VPG):
                    vv = j * VPG + q
                    for ci in range(C // 16):
                        sl = pl.ds(ci * 16, 16)
                        acc = bias_v[sl]
                        if residual:
                            acc = acc + res_v[vv, sl]
                        for k in range(K):
                            acc = acc + rows_v[j, q * KP + k, sl]
                        out_v[vv, sl] = acc
            pltpu.sync_copy(out_v, out_ref.at[pl.ds(vbase, CH)])

    return pl.kernel(
        body,
        out_type=jax.ShapeDtypeStruct((N, C), jnp.float32),
        mesh=mesh,
        scratch_types=scratch,
        compiler_params=pltpu.CompilerParams(use_tc_tiling_on_sc=False),
    )


_gather_plain = _make_gather(residual=False)
_gather_res = _make_gather(residual=True)


def _coeffs(s_c, q_c, gamma, beta):
    cnt = float(NPB * CG)
    sg = s_c.reshape(B, G, CG).sum(-1)
    qg = q_c.reshape(B, G, CG).sum(-1)
    mean_g = sg / cnt
    var_g = qg / cnt - mean_g * mean_g
    rstd_g = lax.rsqrt(var_g + 1e-5)
    mean_c = jnp.repeat(mean_g, CG, axis=1)
    rstd_c = jnp.repeat(rstd_g, CG, axis=1)
    a = rstd_c * gamma[None, :]
    c = beta[None, :] - mean_c * a
    return a, c


def kernel(feats, emb, gamma1, beta1, W1, b1c, We, be, gamma2, beta2, W2,
           b2c, batch_idx, nbrs):
    del batch_idx

    karr = jnp.arange(K, dtype=jnp.int32)[None, :]
    idx = nbrs.astype(jnp.int32).T * K + karr
    idx = jnp.concatenate(
        [idx, jnp.full((N, KP - K), ZIDX, jnp.int32)], axis=1)
    idx2 = idx.reshape(N // VPG, IPG)

    Wcat1 = jnp.transpose(W1, (1, 0, 2)).reshape(C, CK)
    Wcat2 = jnp.transpose(W2, (1, 0, 2)).reshape(C, CK)

    s1, q1 = _stats_call(feats)
    a1, c1 = _coeffs(s1[:, 0, :], q1[:, 0, :], gamma1, beta1)
    Y1 = _matnorm_call(feats, a1[:, None, :], c1[:, None, :], Wcat1)
    h_raw = _gather_plain(Y1.reshape(NP_ROWS * K, C), idx2,
                          jnp.zeros((C,), jnp.float32))

    emb_out = _emb_call(emb, We) + be[None, :]
    t_bc = emb_out + b1c[None, :]

    s2, q2 = _stats_call(h_raw)
    s2c = s2[:, 0, :]
    q2c = q2[:, 0, :]
    s2s = s2c + NPB * t_bc
    q2s = q2c + 2.0 * t_bc * s2c + NPB * t_bc * t_bc
    a2, c2b = _coeffs(s2s, q2s, gamma2, beta2)
    c2 = c2b + t_bc * a2

    Y2 = _matnorm_call(h_raw, a2[:, None, :], c2[:, None, :], Wcat2)
    out = _gather_res(Y2.reshape(NP_ROWS * K, C), idx2, b2c, feats)
    return out


# speedup vs baseline: 19.1505x; 1.4342x over previous
import functools

import jax
import jax.numpy as jnp
from jax import lax
from jax.experimental import pallas as pl
from jax.experimental.pallas import tpu as pltpu
from jax.experimental.pallas import tpu_sc as plsc

N = 200000
C = 64
B = 4
NPB = 50000
G = 32
CG = C // G
K = 27
CK = C * K
TILE = 1000
TPB = NPB // TILE
NT = N // TILE + 1
NP_ROWS = NT * TILE

STILE = 2000
SPB = NPB // STILE

NC = 2
NS = 16
NW = NC * NS
CH = 32
RG = 1
VPG = CH // RG
IPG = VPG * K
PER_W = 6272


def _stats_body(x_ref, s_ref, q_ref):
    t = pl.program_id(1)
    x = x_ref[...]
    s = jnp.sum(x, axis=0)[None, None, :]
    q = jnp.sum(x * x, axis=0)[None, None, :]

    @pl.when(t == 0)
    def _init():
        s_ref[...] = s
        q_ref[...] = q

    @pl.when(t != 0)
    def _acc():
        s_ref[...] += s
        q_ref[...] += q


_stats_call = pl.pallas_call(
    _stats_body,
    grid=(B, SPB),
    in_specs=[pl.BlockSpec((STILE, C), lambda b, t: (b * SPB + t, 0))],
    out_specs=[
        pl.BlockSpec((1, 1, C), lambda b, t: (b, 0, 0)),
        pl.BlockSpec((1, 1, C), lambda b, t: (b, 0, 0)),
    ],
    out_shape=[
        jax.ShapeDtypeStruct((B, 1, C), jnp.float32),
        jax.ShapeDtypeStruct((B, 1, C), jnp.float32),
    ],
)


def _matnorm_body(x_ref, a_ref, c_ref, w_ref, y_ref):
    t = pl.program_id(0)
    x = x_ref[...]
    a = a_ref[0]
    cc = c_ref[0]
    h = x * a + cc
    h = h * jax.nn.sigmoid(h)
    row = t * TILE + lax.broadcasted_iota(jnp.int32, (TILE, 1), 0)
    h = jnp.where(row < N, h, 0.0)
    y_ref[...] = jnp.dot(h, w_ref[...], preferred_element_type=jnp.float32)


_matnorm_call = pl.pallas_call(
    _matnorm_body,
    grid=(NT,),
    in_specs=[
        pl.BlockSpec((TILE, C), lambda t: (jnp.minimum(t, N // TILE - 1), 0)),
        pl.BlockSpec((1, 1, C), lambda t: (jnp.minimum(t // TPB, B - 1), 0, 0)),
        pl.BlockSpec((1, 1, C), lambda t: (jnp.minimum(t // TPB, B - 1), 0, 0)),
        pl.BlockSpec((C, CK), lambda t: (0, 0)),
    ],
    out_specs=pl.BlockSpec((TILE, CK), lambda t: (t, 0)),
    out_shape=jax.ShapeDtypeStruct((NP_ROWS, CK), jnp.float32),
)


def _emb_body(e_ref, w_ref, o_ref):
    e = e_ref[...]
    h = e * jax.nn.sigmoid(e)
    o_ref[...] = jnp.dot(h, w_ref[...], preferred_element_type=jnp.float32)


def _emb_call(emb, We):
    return pl.pallas_call(
        _emb_body,
        out_shape=jax.ShapeDtypeStruct((B, C), jnp.float32),
    )(emb, We)


def _make_gather(residual: bool):
    mesh = plsc.VectorSubcoreMesh(
        core_axis_name="c", subcore_axis_name="s", num_cores=NC,
        num_subcores=NS)

    scratch = [
        pltpu.VMEM((RG, IPG), jnp.int32),
        pltpu.VMEM((RG, IPG, C), jnp.float32),
        pltpu.VMEM((CH, C), jnp.float32),
        pltpu.VMEM((C,), jnp.float32),
        pltpu.VMEM((CH, C), jnp.float32),
        pltpu.SemaphoreType.DMA,
    ]

    def body(tab_ref, idx_ref, bias_ref, *rest):
        if residual:
            res_ref, out_ref, idx_v, rows_v, out_v, bias_v, res_v, sem = rest
        else:
            out_ref, idx_v, rows_v, out_v, bias_v, res_v, sem = rest
        cid = lax.axis_index("c")
        sid = lax.axis_index("s")
        wid = sid * NC + cid
        v0 = wid * PER_W
        rounds = jnp.minimum(PER_W, N - v0) // CH
        pltpu.sync_copy(bias_ref, bias_v)

        @pl.loop(0, rounds)
        def _round(r):
            vbase = pl.multiple_of(v0 + r * CH, CH)
            rbase = vbase // CH
            pltpu.sync_copy(idx_ref.at[rbase], idx_v)
            if residual:
                pltpu.sync_copy(res_ref.at[pl.ds(vbase, CH)], res_v)
            copies = [
                pltpu.async_copy(tab_ref.at[idx_v.at[j]], rows_v.at[j], sem)
                for j in range(RG)
            ]
            for cp in copies:
                cp.wait()
            for j in range(RG):
                for q in range(VPG):
                    vv = j * VPG + q
                    for ci in range(C // 16):
                        sl = pl.ds(ci * 16, 16)
                        acc = bias_v[sl]
                        if residual:
                            acc = acc + res_v[vv, sl]
                        for k in range(K):
                            acc = acc + rows_v[j, q * K + k, sl]
                        out_v[vv, sl] = acc
            pltpu.sync_copy(out_v, out_ref.at[pl.ds(vbase, CH)])

    return pl.kernel(
        body,
        out_type=jax.ShapeDtypeStruct((N, C), jnp.float32),
        mesh=mesh,
        scratch_types=scratch,
        compiler_params=pltpu.CompilerParams(use_tc_tiling_on_sc=False),
    )


_gather_plain = _make_gather(residual=False)
_gather_res = _make_gather(residual=True)


def _coeffs(s_c, q_c, gamma, beta):
    cnt = float(NPB * CG)
    sg = s_c.reshape(B, G, CG).sum(-1)
    qg = q_c.reshape(B, G, CG).sum(-1)
    mean_g = sg / cnt
    var_g = qg / cnt - mean_g * mean_g
    rstd_g = lax.rsqrt(var_g + 1e-5)
    mean_c = jnp.repeat(mean_g, CG, axis=1)
    rstd_c = jnp.repeat(rstd_g, CG, axis=1)
    a = rstd_c * gamma[None, :]
    c = beta[None, :] - mean_c * a
    return a, c


def kernel(feats, emb, gamma1, beta1, W1, b1c, We, be, gamma2, beta2, W2,
           b2c, batch_idx, nbrs):
    del batch_idx

    karr = jnp.arange(K, dtype=jnp.int32)[None, :]
    idx = nbrs.astype(jnp.int32).T * K + karr
    idx2 = idx.reshape(N // CH, RG, IPG)

    Wcat1 = jnp.transpose(W1, (1, 0, 2)).reshape(C, CK)
    Wcat2 = jnp.transpose(W2, (1, 0, 2)).reshape(C, CK)

    s1, q1 = _stats_call(feats)
    a1, c1 = _coeffs(s1[:, 0, :], q1[:, 0, :], gamma1, beta1)
    Y1 = _matnorm_call(feats, a1[:, None, :], c1[:, None, :], Wcat1)
    h_raw = _gather_plain(Y1.reshape(NP_ROWS * K, C), idx2,
                          jnp.zeros((C,), jnp.float32))

    emb_out = _emb_call(emb, We) + be[None, :]
    t_bc = emb_out + b1c[None, :]

    s2, q2 = _stats_call(h_raw)
    s2c = s2[:, 0, :]
    q2c = q2[:, 0, :]
    s2s = s2c + NPB * t_bc
    q2s = q2c + 2.0 * t_bc * s2c + NPB * t_bc * t_bc
    a2, c2b = _coeffs(s2s, q2s, gamma2, beta2)
    c2 = c2b + t_bc * a2

    Y2 = _matnorm_call(h_raw, a2[:, None, :], c2[:, None, :], Wcat2)
    out = _gather_res(Y2.reshape(NP_ROWS * K, C), idx2, b2c, feats)
    return out
